# baseline (speedup 1.0000x reference)
import functools

import jax
import jax.numpy as jnp
from jax import lax
from jax.experimental import pallas as pl
from jax.experimental.pallas import tpu as pltpu
from jax.experimental.pallas import tpu_sc as plsc

T = 2048
D = 768
E = 8
TOP_K = 2
FF = 2048
TM = 512
NT = 15
NTA = 64
NSLOT = NT * TM
FBS = 512
NE9 = float(E + 1)


def _router_body(x_ref, rw_ref, p_ref, w1_ref, w2_ref, eot_ref, nxt_ref,
                 losses_ref):
    x = x_ref[...]
    logits = jnp.dot(x, rw_ref[...], preferred_element_type=jnp.float32)
    iota_e = lax.broadcasted_iota(jnp.int32, (T, E), 1)
    m1 = jnp.max(logits, axis=1, keepdims=True)
    i1 = jnp.min(jnp.where(logits == m1, iota_e, E), axis=1, keepdims=True)
    l2 = jnp.where(iota_e == i1, -jnp.inf, logits)
    m2 = jnp.max(l2, axis=1, keepdims=True)
    i2 = jnp.min(jnp.where(l2 == m2, iota_e, E), axis=1, keepdims=True)
    w1 = 1.0 / (1.0 + jnp.exp(m2 - m1))
    w1_ref[...] = w1
    w2_ref[...] = 1.0 - w1

    e_all = jnp.concatenate([i1, i2], axis=0)
    CS = 1024
    tri = (lax.broadcasted_iota(jnp.int32, (CS, CS), 0) >=
           lax.broadcasted_iota(jnp.int32, (CS, CS), 1)).astype(jnp.bfloat16)
    iota_ce = lax.broadcasted_iota(jnp.int32, (CS, E), 1)
    counts = jnp.zeros((1, E), jnp.float32)
    ranks = []
    for c in range(2 * T // CS):
        e_blk = e_all[c * CS:(c + 1) * CS, :]
        onehot = (e_blk == iota_ce).astype(jnp.bfloat16)
        cum = jnp.dot(tri, onehot, preferred_element_type=jnp.float32)
        ranks.append(jnp.sum(jnp.where(e_blk == iota_ce, cum - 1.0 + counts, 0.0),
                             axis=1, keepdims=True))
        counts = counts + cum[CS - 1:CS, :]
    rank = jnp.concatenate(ranks, axis=0)

    padded = jnp.ceil(counts * (1.0 / TM)) * TM
    iota_e2 = lax.broadcasted_iota(jnp.int32, (2 * T, E), 1)
    start_sel = jnp.sum(jnp.where(iota_e2 < e_all, padded, 0.0),
                        axis=1, keepdims=True)
    p_ref[...] = (start_sel + rank).astype(jnp.int32)

    upper = (lax.broadcasted_iota(jnp.int32, (E, E), 0) <
             lax.broadcasted_iota(jnp.int32, (E, E), 1)).astype(jnp.float32)
    starts = jnp.dot(padded, upper, preferred_element_type=jnp.float32)
    total_padded = jnp.sum(padded)
    pos = lax.broadcasted_iota(jnp.int32, (NTA, 1), 0).astype(jnp.float32) * TM
    cnt = jnp.sum((pos >= starts).astype(jnp.float32), axis=1, keepdims=True)
    eot = jnp.where(pos < total_padded, cnt - 1.0, jnp.float32(E))
    eot_ref[...] = eot.astype(jnp.int32)
    iota_te = lax.broadcasted_iota(jnp.int32, (NTA, E), 1)
    nxt = jnp.min(jnp.where((starts > pos) & (padded > 0.0), iota_te, E),
                  axis=1, keepdims=True)
    nxt_ref[...] = nxt.astype(jnp.int32)

    z_loss = jnp.mean(jnp.sum(logits * logits, axis=1))
    denom = float(T * TOP_K + T)
    ideal = 1.0 / NE9
    d_e = counts / denom - ideal
    d_s = float(T) / denom - ideal
    lb_loss = (jnp.sum(d_e * d_e) + d_s * d_s) / NE9
    total = 0.01 * lb_loss + 0.01 * z_loss
    lane = lax.broadcasted_iota(jnp.int32, (1, 128), 1)
    losses_ref[...] = jnp.where(lane == 0, total,
                       jnp.where(lane == 1, lb_loss,
                        jnp.where(lane == 2, z_loss, 0.0)))


def _ffn_body(eot_ref, nxt_ref, xs_ref, wg_ref, wu_ref, wd_ref, ys_ref,
              wgf_ref, wuf_ref, wdf_ref, wgb_ref, wub_ref, wdb_ref, prev_ref,
              sg, su, sd):
    tt = pl.program_id(0)
    e = eot_ref[tt]

    @pl.when(tt == 0)
    def _():
        prev_ref[0] = -1

    @pl.when(jnp.logical_and(e < E, e != prev_ref[0]))
    def _():
        @pl.when(prev_ref[0] == -1)
        def _():
            pltpu.make_async_copy(wg_ref.at[e], wgf_ref, sg).start()
            pltpu.make_async_copy(wu_ref.at[e], wuf_ref, su).start()
            pltpu.make_async_copy(wd_ref.at[e], wdf_ref, sd).start()

        pltpu.make_async_copy(wg_ref.at[e], wgf_ref, sg).wait()
        pltpu.make_async_copy(wu_ref.at[e], wuf_ref, su).wait()
        pltpu.make_async_copy(wd_ref.at[e], wdf_ref, sd).wait()
        wgb_ref[...] = wgf_ref[...].astype(jnp.bfloat16)
        wub_ref[...] = wuf_ref[...].astype(jnp.bfloat16)
        wdb_ref[...] = wdf_ref[...].astype(jnp.bfloat16)
        nx = nxt_ref[tt]

        @pl.when(nx < E)
        def _():
            pltpu.make_async_copy(wg_ref.at[nx], wgf_ref, sg).start()
            pltpu.make_async_copy(wu_ref.at[nx], wuf_ref, su).start()
            pltpu.make_async_copy(wd_ref.at[nx], wdf_ref, sd).start()

        prev_ref[0] = e

    @pl.when(e < E)
    def _():
        xb = xs_ref[...].astype(jnp.bfloat16)
        g = jnp.dot(xb, wgb_ref[...], preferred_element_type=jnp.float32)
        u = jnp.dot(xb, wub_ref[...], preferred_element_type=jnp.float32)
        h = (g * (1.0 / (1.0 + jnp.exp(-g))) * u).astype(jnp.bfloat16)
        ys_ref[...] = jnp.dot(h, wdb_ref[...], preferred_element_type=jnp.float32)


def _shared_body(x_ref, swg_ref, swu_ref, swd_ref, g1_ref, g2_ref,
                 w1_ref, w2_ref, out_ref):
    f = pl.program_id(0)

    @pl.when(f == 0)
    def _():
        out_ref[...] = (w1_ref[...] * g1_ref[...]
                        + w2_ref[...] * g2_ref[...])

    xb = x_ref[...].astype(jnp.bfloat16)
    g = jnp.dot(xb, swg_ref[...].astype(jnp.bfloat16),
                preferred_element_type=jnp.float32)
    u = jnp.dot(xb, swu_ref[...].astype(jnp.bfloat16),
                preferred_element_type=jnp.float32)
    h = (g * (1.0 / (1.0 + jnp.exp(-g))) * u).astype(jnp.bfloat16)
    out_ref[...] += jnp.dot(h, swd_ref[...].astype(jnp.bfloat16),
                            preferred_element_type=jnp.float32)


def _sc_dispatch_call(x, p_flat):
    mesh = plsc.VectorSubcoreMesh(core_axis_name="c", subcore_axis_name="s")
    NW = 32
    CH = (2 * T) // NW

    @functools.partial(
        pl.kernel,
        out_type=jax.ShapeDtypeStruct((NSLOT, D), jnp.float32),
        mesh=mesh,
        scratch_types=[pltpu.VMEM((CH,), jnp.int32),
                       pltpu.VMEM((CH, D), jnp.float32),
                       pltpu.SemaphoreType.DMA],
    )
    def run(x_hbm, p_hbm, xs_hbm, idx_v, rows_v, sem):
        wid = lax.axis_index("s") * 2 + lax.axis_index("c")
        base = wid * CH
        tok = lax.rem(base, T)
        pltpu.sync_copy(p_hbm.at[pl.ds(base, CH)], idx_v)
        pltpu.sync_copy(x_hbm.at[pl.ds(tok, CH)], rows_v)
        pltpu.async_copy(rows_v, xs_hbm.at[idx_v], sem).wait()

    return run(x, p_flat)


def _sc_gather_call(ys, p_flat):
    mesh = plsc.VectorSubcoreMesh(core_axis_name="c", subcore_axis_name="s")
    NW = 32
    CH = (2 * T) // NW

    @functools.partial(
        pl.kernel,
        out_type=jax.ShapeDtypeStruct((2 * T, D), jnp.float32),
        mesh=mesh,
        scratch_types=[pltpu.VMEM((CH,), jnp.int32),
                       pltpu.VMEM((CH, D), jnp.float32),
                       pltpu.SemaphoreType.DMA],
    )
    def run(ys_hbm, p_hbm, g_hbm, idx_v, rows_v, sem):
        wid = lax.axis_index("s") * 2 + lax.axis_index("c")
        base = wid * CH
        pltpu.sync_copy(p_hbm.at[pl.ds(base, CH)], idx_v)
        pltpu.async_copy(ys_hbm.at[idx_v], rows_v, sem).wait()
        pltpu.sync_copy(rows_v, g_hbm.at[pl.ds(base, CH)])

    return run(ys, p_flat)


def kernel(hidden_states, router_W, Wg, Wu, Wd, sWg, sWu, sWd):
    B, S, _ = hidden_states.shape
    x = hidden_states.reshape(T, D)

    p_all, w1, w2, eot64, nxt64, losses = pl.pallas_call(
        _router_body,
        out_shape=(
            jax.ShapeDtypeStruct((2 * T, 1), jnp.int32),
            jax.ShapeDtypeStruct((T, 1), jnp.float32),
            jax.ShapeDtypeStruct((T, 1), jnp.float32),
            jax.ShapeDtypeStruct((NTA, 1), jnp.int32),
            jax.ShapeDtypeStruct((NTA, 1), jnp.int32),
            jax.ShapeDtypeStruct((1, 128), jnp.float32),
        ),
    )(x, router_W)
    p_flat = p_all.reshape(2 * T)
    eot = eot64.reshape(NTA)
    nxt = nxt64.reshape(NTA)

    xs = _sc_dispatch_call(x, p_flat)

    grid_spec = pltpu.PrefetchScalarGridSpec(
        num_scalar_prefetch=2,
        grid=(NT,),
        in_specs=[
            pl.BlockSpec((TM, D), lambda tt, eot_r, nxt_r: (tt, 0)),
            pl.BlockSpec(memory_space=pltpu.MemorySpace.HBM),
            pl.BlockSpec(memory_space=pltpu.MemorySpace.HBM),
            pl.BlockSpec(memory_space=pltpu.MemorySpace.HBM),
        ],
        out_specs=pl.BlockSpec((TM, D), lambda tt, eot_r, nxt_r: (tt, 0)),
        scratch_shapes=[
            pltpu.VMEM((D, FF), jnp.float32),
            pltpu.VMEM((D, FF), jnp.float32),
            pltpu.VMEM((FF, D), jnp.float32),
            pltpu.VMEM((D, FF), jnp.bfloat16),
            pltpu.VMEM((D, FF), jnp.bfloat16),
            pltpu.VMEM((FF, D), jnp.bfloat16),
            pltpu.SMEM((1,), jnp.int32),
            pltpu.SemaphoreType.DMA,
            pltpu.SemaphoreType.DMA,
            pltpu.SemaphoreType.DMA,
        ],
    )
    ys = pl.pallas_call(
        _ffn_body,
        grid_spec=grid_spec,
        out_shape=jax.ShapeDtypeStruct((NSLOT, D), jnp.float32),
        compiler_params=pltpu.CompilerParams(
            dimension_semantics=("arbitrary",),
        ),
    )(eot, nxt, xs, Wg, Wu, Wd)

    g = _sc_gather_call(ys, p_flat)

    nfs = FF // FBS
    out = pl.pallas_call(
        _shared_body,
        grid=(nfs,),
        in_specs=[
            pl.BlockSpec((T, D), lambda f: (0, 0)),
            pl.BlockSpec((D, FBS), lambda f: (0, f)),
            pl.BlockSpec((D, FBS), lambda f: (0, f)),
            pl.BlockSpec((FBS, D), lambda f: (f, 0)),
            pl.BlockSpec((T, D), lambda f: (0, 0)),
            pl.BlockSpec((T, D), lambda f: (1, 0)),
            pl.BlockSpec((T, 1), lambda f: (0, 0)),
            pl.BlockSpec((T, 1), lambda f: (0, 0)),
        ],
        out_specs=pl.BlockSpec((T, D), lambda f: (0, 0)),
        out_shape=jax.ShapeDtypeStruct((T, D), jnp.float32),
        compiler_params=pltpu.CompilerParams(
            dimension_semantics=("arbitrary",),
        ),
    )(x, sWg, sWu, sWd, g, g, w1, w2)

    final = out.reshape(B, S, D)
    return (final, losses[0, 0], losses[0, 1], losses[0, 2])


# speedup vs baseline: 1.0348x; 1.0348x over previous
import functools

import jax
import jax.numpy as jnp
from jax import lax
from jax.experimental import pallas as pl
from jax.experimental.pallas import tpu as pltpu
from jax.experimental.pallas import tpu_sc as plsc

T = 2048
D = 768
E = 8
TOP_K = 2
FF = 2048
TM = 256
NT = 24
NTA = 64
NSLOT = NT * TM
FBS = 512
NE9 = float(E + 1)


def _router_body(x_ref, rw_ref, p_ref, w1_ref, w2_ref, eot_ref, nxt_ref,
                 losses_ref):
    x = x_ref[...]
    logits = jnp.dot(x, rw_ref[...], preferred_element_type=jnp.float32)
    iota_e = lax.broadcasted_iota(jnp.int32, (T, E), 1)
    m1 = jnp.max(logits, axis=1, keepdims=True)
    i1 = jnp.min(jnp.where(logits == m1, iota_e, E), axis=1, keepdims=True)
    l2 = jnp.where(iota_e == i1, -jnp.inf, logits)
    m2 = jnp.max(l2, axis=1, keepdims=True)
    i2 = jnp.min(jnp.where(l2 == m2, iota_e, E), axis=1, keepdims=True)
    w1 = 1.0 / (1.0 + jnp.exp(m2 - m1))
    w1_ref[...] = w1
    w2_ref[...] = 1.0 - w1

    e_all = jnp.concatenate([i1, i2], axis=0)
    CS = 1024
    tri = (lax.broadcasted_iota(jnp.int32, (CS, CS), 0) >=
           lax.broadcasted_iota(jnp.int32, (CS, CS), 1)).astype(jnp.bfloat16)
    iota_ce = lax.broadcasted_iota(jnp.int32, (CS, E), 1)
    counts = jnp.zeros((1, E), jnp.float32)
    ranks = []
    for c in range(2 * T // CS):
        e_blk = e_all[c * CS:(c + 1) * CS, :]
        onehot = (e_blk == iota_ce).astype(jnp.bfloat16)
        cum = jnp.dot(tri, onehot, preferred_element_type=jnp.float32)
        ranks.append(jnp.sum(jnp.where(e_blk == iota_ce, cum - 1.0 + counts, 0.0),
                             axis=1, keepdims=True))
        counts = counts + cum[CS - 1:CS, :]
    rank = jnp.concatenate(ranks, axis=0)

    padded = jnp.ceil(counts * (1.0 / TM)) * TM
    iota_e2 = lax.broadcasted_iota(jnp.int32, (2 * T, E), 1)
    start_sel = jnp.sum(jnp.where(iota_e2 < e_all, padded, 0.0),
                        axis=1, keepdims=True)
    p_ref[...] = (start_sel + rank).astype(jnp.int32)

    upper = (lax.broadcasted_iota(jnp.int32, (E, E), 0) <
             lax.broadcasted_iota(jnp.int32, (E, E), 1)).astype(jnp.float32)
    starts = jnp.dot(padded, upper, preferred_element_type=jnp.float32)
    total_padded = jnp.sum(padded)
    pos = lax.broadcasted_iota(jnp.int32, (NTA, 1), 0).astype(jnp.float32) * TM
    cnt = jnp.sum((pos >= starts).astype(jnp.float32), axis=1, keepdims=True)
    eot = jnp.where(pos < total_padded, cnt - 1.0, jnp.float32(E))
    eot_ref[...] = eot.astype(jnp.int32)
    iota_te = lax.broadcasted_iota(jnp.int32, (NTA, E), 1)
    nxt = jnp.min(jnp.where((starts > pos) & (padded > 0.0), iota_te, E),
                  axis=1, keepdims=True)
    nxt_ref[...] = nxt.astype(jnp.int32)

    z_loss = jnp.mean(jnp.sum(logits * logits, axis=1))
    denom = float(T * TOP_K + T)
    ideal = 1.0 / NE9
    d_e = counts / denom - ideal
    d_s = float(T) / denom - ideal
    lb_loss = (jnp.sum(d_e * d_e) + d_s * d_s) / NE9
    total = 0.01 * lb_loss + 0.01 * z_loss
    lane = lax.broadcasted_iota(jnp.int32, (1, 128), 1)
    losses_ref[...] = jnp.where(lane == 0, total,
                       jnp.where(lane == 1, lb_loss,
                        jnp.where(lane == 2, z_loss, 0.0)))


def _ffn_body(eot_ref, nxt_ref, xs_ref, wg_ref, wu_ref, wd_ref, ys_ref,
              wgf_ref, wuf_ref, wdf_ref, wgb_ref, wub_ref, wdb_ref, prev_ref,
              sg, su, sd):
    tt = pl.program_id(0)
    e = eot_ref[tt]

    @pl.when(tt == 0)
    def _():
        prev_ref[0] = -1

    @pl.when(jnp.logical_and(e < E, e != prev_ref[0]))
    def _():
        @pl.when(prev_ref[0] == -1)
        def _():
            pltpu.make_async_copy(wg_ref.at[e], wgf_ref, sg).start()
            pltpu.make_async_copy(wu_ref.at[e], wuf_ref, su).start()
            pltpu.make_async_copy(wd_ref.at[e], wdf_ref, sd).start()

        pltpu.make_async_copy(wg_ref.at[e], wgf_ref, sg).wait()
        pltpu.make_async_copy(wu_ref.at[e], wuf_ref, su).wait()
        pltpu.make_async_copy(wd_ref.at[e], wdf_ref, sd).wait()
        wgb_ref[...] = wgf_ref[...].astype(jnp.bfloat16)
        wub_ref[...] = wuf_ref[...].astype(jnp.bfloat16)
        wdb_ref[...] = wdf_ref[...].astype(jnp.bfloat16)
        nx = nxt_ref[tt]

        @pl.when(nx < E)
        def _():
            pltpu.make_async_copy(wg_ref.at[nx], wgf_ref, sg).start()
            pltpu.make_async_copy(wu_ref.at[nx], wuf_ref, su).start()
            pltpu.make_async_copy(wd_ref.at[nx], wdf_ref, sd).start()

        prev_ref[0] = e

    @pl.when(e < E)
    def _():
        xb = xs_ref[...].astype(jnp.bfloat16)
        g = jnp.dot(xb, wgb_ref[...], preferred_element_type=jnp.float32)
        u = jnp.dot(xb, wub_ref[...], preferred_element_type=jnp.float32)
        h = (g * (1.0 / (1.0 + jnp.exp(-g))) * u).astype(jnp.bfloat16)
        ys_ref[...] = jnp.dot(h, wdb_ref[...], preferred_element_type=jnp.float32)


def _shared_body(x_ref, swg_ref, swu_ref, swd_ref, g1_ref, g2_ref,
                 w1_ref, w2_ref, out_ref):
    f = pl.program_id(0)

    @pl.when(f == 0)
    def _():
        out_ref[...] = (w1_ref[...] * g1_ref[...]
                        + w2_ref[...] * g2_ref[...])

    xb = x_ref[...].astype(jnp.bfloat16)
    g = jnp.dot(xb, swg_ref[...].astype(jnp.bfloat16),
                preferred_element_type=jnp.float32)
    u = jnp.dot(xb, swu_ref[...].astype(jnp.bfloat16),
                preferred_element_type=jnp.float32)
    h = (g * (1.0 / (1.0 + jnp.exp(-g))) * u).astype(jnp.bfloat16)
    out_ref[...] += jnp.dot(h, swd_ref[...].astype(jnp.bfloat16),
                            preferred_element_type=jnp.float32)


def _sc_dispatch_call(x, p_flat):
    mesh = plsc.VectorSubcoreMesh(core_axis_name="c", subcore_axis_name="s")
    NW = 32
    CH = (2 * T) // NW

    @functools.partial(
        pl.kernel,
        out_type=jax.ShapeDtypeStruct((NSLOT, D), jnp.float32),
        mesh=mesh,
        scratch_types=[pltpu.VMEM((CH,), jnp.int32),
                       pltpu.VMEM((CH, D), jnp.float32),
                       pltpu.SemaphoreType.DMA],
    )
    def run(x_hbm, p_hbm, xs_hbm, idx_v, rows_v, sem):
        wid = lax.axis_index("s") * 2 + lax.axis_index("c")
        base = wid * CH
        tok = lax.rem(base, T)
        pltpu.sync_copy(p_hbm.at[pl.ds(base, CH)], idx_v)
        pltpu.sync_copy(x_hbm.at[pl.ds(tok, CH)], rows_v)
        pltpu.async_copy(rows_v, xs_hbm.at[idx_v], sem).wait()

    return run(x, p_flat)


def _sc_gather_call(ys, p_flat):
    mesh = plsc.VectorSubcoreMesh(core_axis_name="c", subcore_axis_name="s")
    NW = 32
    CH = (2 * T) // NW

    @functools.partial(
        pl.kernel,
        out_type=jax.ShapeDtypeStruct((2 * T, D), jnp.float32),
        mesh=mesh,
        scratch_types=[pltpu.VMEM((CH,), jnp.int32),
                       pltpu.VMEM((CH, D), jnp.float32),
                       pltpu.SemaphoreType.DMA],
    )
    def run(ys_hbm, p_hbm, g_hbm, idx_v, rows_v, sem):
        wid = lax.axis_index("s") * 2 + lax.axis_index("c")
        base = wid * CH
        pltpu.sync_copy(p_hbm.at[pl.ds(base, CH)], idx_v)
        pltpu.async_copy(ys_hbm.at[idx_v], rows_v, sem).wait()
        pltpu.sync_copy(rows_v, g_hbm.at[pl.ds(base, CH)])

    return run(ys, p_flat)


def kernel(hidden_states, router_W, Wg, Wu, Wd, sWg, sWu, sWd):
    B, S, _ = hidden_states.shape
    x = hidden_states.reshape(T, D)

    p_all, w1, w2, eot64, nxt64, losses = pl.pallas_call(
        _router_body,
        out_shape=(
            jax.ShapeDtypeStruct((2 * T, 1), jnp.int32),
            jax.ShapeDtypeStruct((T, 1), jnp.float32),
            jax.ShapeDtypeStruct((T, 1), jnp.float32),
            jax.ShapeDtypeStruct((NTA, 1), jnp.int32),
            jax.ShapeDtypeStruct((NTA, 1), jnp.int32),
            jax.ShapeDtypeStruct((1, 128), jnp.float32),
        ),
    )(x, router_W)
    p_flat = p_all.reshape(2 * T)
    eot = eot64.reshape(NTA)
    nxt = nxt64.reshape(NTA)

    xs = _sc_dispatch_call(x, p_flat)

    grid_spec = pltpu.PrefetchScalarGridSpec(
        num_scalar_prefetch=2,
        grid=(NT,),
        in_specs=[
            pl.BlockSpec((TM, D), lambda tt, eot_r, nxt_r: (tt, 0)),
            pl.BlockSpec(memory_space=pltpu.MemorySpace.HBM),
            pl.BlockSpec(memory_space=pltpu.MemorySpace.HBM),
            pl.BlockSpec(memory_space=pltpu.MemorySpace.HBM),
        ],
        out_specs=pl.BlockSpec((TM, D), lambda tt, eot_r, nxt_r: (tt, 0)),
        scratch_shapes=[
            pltpu.VMEM((D, FF), jnp.float32),
            pltpu.VMEM((D, FF), jnp.float32),
            pltpu.VMEM((FF, D), jnp.float32),
            pltpu.VMEM((D, FF), jnp.bfloat16),
            pltpu.VMEM((D, FF), jnp.bfloat16),
            pltpu.VMEM((FF, D), jnp.bfloat16),
            pltpu.SMEM((1,), jnp.int32),
            pltpu.SemaphoreType.DMA,
            pltpu.SemaphoreType.DMA,
            pltpu.SemaphoreType.DMA,
        ],
    )
    ys = pl.pallas_call(
        _ffn_body,
        grid_spec=grid_spec,
        out_shape=jax.ShapeDtypeStruct((NSLOT, D), jnp.float32),
        compiler_params=pltpu.CompilerParams(
            dimension_semantics=("arbitrary",),
        ),
    )(eot, nxt, xs, Wg, Wu, Wd)

    g = _sc_gather_call(ys, p_flat)

    nfs = FF // FBS
    out = pl.pallas_call(
        _shared_body,
        grid=(nfs,),
        in_specs=[
            pl.BlockSpec((T, D), lambda f: (0, 0)),
            pl.BlockSpec((D, FBS), lambda f: (0, f)),
            pl.BlockSpec((D, FBS), lambda f: (0, f)),
            pl.BlockSpec((FBS, D), lambda f: (f, 0)),
            pl.BlockSpec((T, D), lambda f: (0, 0)),
            pl.BlockSpec((T, D), lambda f: (1, 0)),
            pl.BlockSpec((T, 1), lambda f: (0, 0)),
            pl.BlockSpec((T, 1), lambda f: (0, 0)),
        ],
        out_specs=pl.BlockSpec((T, D), lambda f: (0, 0)),
        out_shape=jax.ShapeDtypeStruct((T, D), jnp.float32),
        compiler_params=pltpu.CompilerParams(
            dimension_semantics=("arbitrary",),
        ),
    )(x, sWg, sWu, sWd, g, g, w1, w2)

    final = out.reshape(B, S, D)
    return (final, losses[0, 0], losses[0, 1], losses[0, 2])
